# baseline (speedup 1.0000x reference)
import functools

import jax
import jax.numpy as jnp
import numpy as np
from jax.experimental import pallas as pl
from jax.experimental.pallas import tpu as pltpu

_CAP_FACTOR = 1.0
_RB = 512
_PAD = 128


def _gelu_exact(z):
    return 0.5 * z * (1.0 + jax.lax.erf(z * np.float32(1.0 / np.sqrt(2.0))))


def _router_kernel(xb_ref, wg_ref, w1_ref, w2_ref, out_ref,
                   wall_ref, wallt_ref, ptall_ref, accall_ref,
                   sel_ref, acc_ref, *, E, C, HB):
    e = pl.program_id(1)
    hb = pl.program_id(2)
    T, D = xb_ref.shape

    @pl.when((e == 0) & (hb == 0))
    def _init():
        s = jnp.dot(xb_ref[...], wg_ref[...],
                    preferred_element_type=jnp.float32)
        m = jnp.max(s, axis=0, keepdims=True)
        ex = jnp.exp(s - m)
        wall = ex / jnp.sum(ex, axis=0, keepdims=True)
        wall_ref[...] = wall
        wallt_ref[...] = wall.T

    mrow = jax.lax.broadcasted_iota(jnp.int32, (1, E), 1) == e
    w_col = jnp.sum(jnp.where(mrow, wall_ref[...], 0.0),
                    axis=1, keepdims=True)

    @pl.when(hb == 0)
    def _route():
        mcol = jax.lax.broadcasted_iota(jnp.int32, (E, 1), 0) == e
        w_row = jnp.sum(jnp.where(mcol, wallt_ref[...], 0.0),
                        axis=0, keepdims=True)

        irow = jax.lax.broadcasted_iota(jnp.int32, (1, T), 1)
        parts = []
        for k in range(T // _RB):
            wj = w_col[k * _RB:(k + 1) * _RB]
            jj = k * _RB + jax.lax.broadcasted_iota(jnp.int32, (_RB, 1), 0)
            ind = ((wj > w_row) | ((wj == w_row) & (jj < irow))
                   ).astype(jnp.float32)
            parts.append(jnp.sum(ind, axis=0, keepdims=True))
        rank = sum(parts).astype(jnp.int32)

        ptT = (jax.lax.broadcasted_iota(jnp.int32, (C, 1), 0) == rank
               ).astype(jnp.bfloat16)
        ptall_ref[pl.ds(e * C, C), :] = ptT

        sel_ref[...] = jnp.dot(
            ptT, xb_ref[...],
            preferred_element_type=jnp.float32).astype(jnp.bfloat16)
        acc_ref[...] = jnp.zeros_like(acc_ref)

    z = jnp.dot(sel_ref[...], w1_ref[...],
                preferred_element_type=jnp.float32)
    h = _gelu_exact(z).astype(jnp.bfloat16)
    acc_ref[...] += jnp.dot(h, w2_ref[...],
                            preferred_element_type=jnp.float32)

    @pl.when(hb == HB - 1)
    def _slot_out():
        hi = w_col.astype(jnp.bfloat16)
        lo = (w_col - hi.astype(jnp.float32)).astype(jnp.bfloat16)
        ptT = ptall_ref[pl.ds(e * C, C), :]
        vals = (jnp.dot(ptT, hi, preferred_element_type=jnp.float32)
                + jnp.dot(ptT, lo, preferred_element_type=jnp.float32))
        weighted = (acc_ref[...] * vals).astype(jnp.bfloat16)
        npad = jnp.where(
            jax.lax.broadcasted_iota(jnp.int32, (C, _PAD), 1) == 0,
            vals, 0.0).astype(jnp.bfloat16)
        accall_ref[pl.ds(e * C, C), :] = jnp.concatenate(
            [weighted, npad], axis=1)

    @pl.when((e == E - 1) & (hb == HB - 1))
    def _scatter():
        outa = jax.lax.dot_general(
            ptall_ref[...], accall_ref[...], (((0,), (0,)), ((), ())),
            preferred_element_type=jnp.float32)
        norm = jnp.maximum(outa[:, D:D + 1], 1e-8)
        out_ref[...] = outa[:, :D] / norm


def _forward(x, Wg, W1, W2, interpret=False):
    B, T, D = x.shape
    E = Wg.shape[1]
    H = W1.shape[2]
    C = min(T, max(1, int(T * _CAP_FACTOR / E)))
    HB = 8
    HBLK = H // HB
    xb = x.astype(jnp.bfloat16)
    wgb = Wg.astype(jnp.bfloat16)
    w1b = W1.astype(jnp.bfloat16)
    w2b = W2.astype(jnp.bfloat16)
    return pl.pallas_call(
        functools.partial(_router_kernel, E=E, C=C, HB=HB),
        grid=(B, E, HB),
        in_specs=[
            pl.BlockSpec((None, T, D), lambda b, e, hb: (b, 0, 0)),
            pl.BlockSpec((D, E), lambda b, e, hb: (0, 0)),
            pl.BlockSpec((None, D, HBLK), lambda b, e, hb: (e, 0, hb)),
            pl.BlockSpec((None, HBLK, D), lambda b, e, hb: (e, hb, 0)),
        ],
        out_specs=pl.BlockSpec((None, T, D), lambda b, e, hb: (b, 0, 0)),
        out_shape=jax.ShapeDtypeStruct((B, T, D), jnp.float32),
        scratch_shapes=[
            pltpu.VMEM((T, E), jnp.float32),
            pltpu.VMEM((E, T), jnp.float32),
            pltpu.VMEM((E * C, T), jnp.bfloat16),
            pltpu.VMEM((E * C, D + _PAD), jnp.bfloat16),
            pltpu.VMEM((C, D), jnp.bfloat16),
            pltpu.VMEM((C, D), jnp.float32),
        ],
        compiler_params=pltpu.CompilerParams(
            vmem_limit_bytes=64 * 1024 * 1024),
        interpret=interpret,
    )(xb, wgb, w1b, w2b)


def kernel(x, Wg, W1, W2):
    return _forward(x, Wg, W1, W2)


# speedup vs baseline: 1.1778x; 1.1778x over previous
import functools

import jax
import jax.numpy as jnp
import numpy as np
from jax.experimental import pallas as pl
from jax.experimental.pallas import tpu as pltpu

_CAP_FACTOR = 1.0
_RB = 512
_PAD = 128


def _gelu_exact(z):
    return 0.5 * z * (1.0 + jax.lax.erf(z * np.float32(1.0 / np.sqrt(2.0))))


def _router_kernel(xb_ref, wg_ref, w1_ref, w2_ref, out_ref,
                   wall_ref, wallt_ref, ptall_ref, accall_ref,
                   sel_ref, acc_ref, *, E, C, HB):
    e = pl.program_id(1)
    hb = pl.program_id(2)
    T, D = xb_ref.shape

    @pl.when((e == 0) & (hb == 0))
    def _init():
        s = jnp.dot(xb_ref[...], wg_ref[...],
                    preferred_element_type=jnp.float32)
        m = jnp.max(s, axis=0, keepdims=True)
        ex = jnp.exp(s - m)
        wall = ex / jnp.sum(ex, axis=0, keepdims=True)
        wall_ref[...] = wall
        wallt_ref[...] = wall.T

    def _w_col():
        mrow = jax.lax.broadcasted_iota(jnp.int32, (1, E), 1) == e
        return jnp.sum(jnp.where(mrow, wall_ref[...], 0.0),
                       axis=1, keepdims=True)

    @pl.when(hb == 0)
    def _route():
        w_col = _w_col()
        mcol = jax.lax.broadcasted_iota(jnp.int32, (E, 1), 0) == e
        w_row = jnp.sum(jnp.where(mcol, wallt_ref[...], 0.0),
                        axis=0, keepdims=True)

        irow = jax.lax.broadcasted_iota(jnp.int32, (1, T), 1)
        parts = []
        for k in range(T // _RB):
            wj = w_col[k * _RB:(k + 1) * _RB]
            jj = k * _RB + jax.lax.broadcasted_iota(jnp.int32, (_RB, 1), 0)
            ind = ((wj > w_row) | ((wj == w_row) & (jj < irow))
                   ).astype(jnp.float32)
            parts.append(jnp.sum(ind, axis=0, keepdims=True))
        rank = sum(parts).astype(jnp.int32)

        ptT = (jax.lax.broadcasted_iota(jnp.int32, (C, 1), 0) == rank
               ).astype(jnp.bfloat16)
        ptall_ref[pl.ds(e * C, C), :] = ptT

        sel_ref[...] = jnp.dot(
            ptT, xb_ref[...],
            preferred_element_type=jnp.float32).astype(jnp.bfloat16)
        acc_ref[...] = jnp.zeros_like(acc_ref)

    z = jnp.dot(sel_ref[...], w1_ref[...],
                preferred_element_type=jnp.float32)
    h = _gelu_exact(z).astype(jnp.bfloat16)
    acc_ref[...] += jnp.dot(h, w2_ref[...],
                            preferred_element_type=jnp.float32)

    @pl.when(hb == HB - 1)
    def _slot_out():
        w_col = _w_col()
        hi = w_col.astype(jnp.bfloat16)
        lo = (w_col - hi.astype(jnp.float32)).astype(jnp.bfloat16)
        ptT = ptall_ref[pl.ds(e * C, C), :]
        vals = (jnp.dot(ptT, hi, preferred_element_type=jnp.float32)
                + jnp.dot(ptT, lo, preferred_element_type=jnp.float32))
        weighted = (acc_ref[...] * vals).astype(jnp.bfloat16)
        npad = jnp.where(
            jax.lax.broadcasted_iota(jnp.int32, (C, _PAD), 1) == 0,
            vals, 0.0).astype(jnp.bfloat16)
        accall_ref[pl.ds(e * C, C), :] = jnp.concatenate(
            [weighted, npad], axis=1)

    @pl.when((e == E - 1) & (hb == HB - 1))
    def _scatter():
        outa = jax.lax.dot_general(
            ptall_ref[...], accall_ref[...], (((0,), (0,)), ((), ())),
            preferred_element_type=jnp.float32)
        norm = jnp.maximum(outa[:, D:D + 1], 1e-8)
        out_ref[...] = outa[:, :D] / norm


def _forward(x, Wg, W1, W2, interpret=False):
    B, T, D = x.shape
    E = Wg.shape[1]
    H = W1.shape[2]
    C = min(T, max(1, int(T * _CAP_FACTOR / E)))
    HB = 4
    HBLK = H // HB
    xb = x.astype(jnp.bfloat16)
    wgb = Wg.astype(jnp.bfloat16)
    w1b = W1.astype(jnp.bfloat16)
    w2b = W2.astype(jnp.bfloat16)
    return pl.pallas_call(
        functools.partial(_router_kernel, E=E, C=C, HB=HB),
        grid=(B, E, HB),
        in_specs=[
            pl.BlockSpec((None, T, D), lambda b, e, hb: (b, 0, 0)),
            pl.BlockSpec((D, E), lambda b, e, hb: (0, 0)),
            pl.BlockSpec((None, D, HBLK), lambda b, e, hb: (e, 0, hb)),
            pl.BlockSpec((None, HBLK, D), lambda b, e, hb: (e, hb, 0)),
        ],
        out_specs=pl.BlockSpec((None, T, D), lambda b, e, hb: (b, 0, 0)),
        out_shape=jax.ShapeDtypeStruct((B, T, D), jnp.float32),
        scratch_shapes=[
            pltpu.VMEM((T, E), jnp.float32),
            pltpu.VMEM((E, T), jnp.float32),
            pltpu.VMEM((E * C, T), jnp.bfloat16),
            pltpu.VMEM((E * C, D + _PAD), jnp.bfloat16),
            pltpu.VMEM((C, D), jnp.bfloat16),
            pltpu.VMEM((C, D), jnp.float32),
        ],
        compiler_params=pltpu.CompilerParams(
            vmem_limit_bytes=64 * 1024 * 1024),
        interpret=interpret,
    )(xb, wgb, w1b, w2b)


def kernel(x, Wg, W1, W2):
    return _forward(x, Wg, W1, W2)


# speedup vs baseline: 1.2040x; 1.0222x over previous
import functools

import jax
import jax.numpy as jnp
import numpy as np
from jax.experimental import pallas as pl
from jax.experimental.pallas import tpu as pltpu

_CAP_FACTOR = 1.0
_RB = 512


def _gelu_exact(z):
    return 0.5 * z * (1.0 + jax.lax.erf(z * np.float32(1.0 / np.sqrt(2.0))))


def _router_kernel(xb_ref, wg_ref, w1_ref, w2_ref, out_ref,
                   wall_ref, wallt_ref, tp_ref, pt_ref, wsel_ref,
                   sel_ref, acc_ref, *, E, C, HB):
    e = pl.program_id(1)
    hb = pl.program_id(2)
    T, D = xb_ref.shape

    @pl.when((e == 0) & (hb == 0))
    def _init():
        s = jnp.dot(xb_ref[...], wg_ref[...],
                    preferred_element_type=jnp.float32)
        m = jnp.max(s, axis=0, keepdims=True)
        ex = jnp.exp(s - m)
        wall = ex / jnp.sum(ex, axis=0, keepdims=True)
        wall_ref[...] = wall
        wallt_ref[...] = wall.T

    @pl.when(hb == 0)
    def _route():
        mrow = jax.lax.broadcasted_iota(jnp.int32, (1, E), 1) == e
        w_col = jnp.sum(jnp.where(mrow, wall_ref[...], 0.0),
                        axis=1, keepdims=True)
        mcol = jax.lax.broadcasted_iota(jnp.int32, (E, 1), 0) == e
        w_row = jnp.sum(jnp.where(mcol, wallt_ref[...], 0.0),
                        axis=0, keepdims=True)

        jrow = jax.lax.broadcasted_iota(jnp.int32, (1, T), 1)
        cnts = []
        for k in range(T // _RB):
            wi = w_col[k * _RB:(k + 1) * _RB]
            ii = k * _RB + jax.lax.broadcasted_iota(jnp.int32, (_RB, 1), 0)
            ind = ((w_row > wi) | ((w_row == wi) & (jrow < ii))
                   ).astype(jnp.float32)
            cnts.append(jnp.sum(ind, axis=1, keepdims=True))
        rank = jnp.concatenate(cnts, axis=0).astype(jnp.int32)
        wsel_ref[...] = jnp.where(rank < C, w_col, 0.0)

        iota_c = jax.lax.broadcasted_iota(jnp.int32, (T, C), 1)
        pt = (rank == iota_c).astype(jnp.bfloat16)
        pt_ref[...] = pt

        sel_ref[...] = jax.lax.dot_general(
            pt, xb_ref[...], (((0,), (0,)), ((), ())),
            preferred_element_type=jnp.float32).astype(jnp.bfloat16)
        acc_ref[...] = jnp.zeros_like(acc_ref)

    z = jnp.dot(sel_ref[...], w1_ref[...],
                preferred_element_type=jnp.float32)
    h = _gelu_exact(z).astype(jnp.bfloat16)
    acc_ref[...] += jnp.dot(h, w2_ref[...],
                            preferred_element_type=jnp.float32)

    @pl.when(hb == HB - 1)
    def _combine():
        wsel = wsel_ref[...]
        contrib = jnp.dot(pt_ref[...], acc_ref[...].astype(jnp.bfloat16),
                          preferred_element_type=jnp.float32)

        @pl.when(e == 0)
        def _first():
            out_ref[...] = contrib * wsel
            tp_ref[...] = wsel

        @pl.when(e != 0)
        def _rest():
            out_ref[...] += contrib * wsel
            tp_ref[...] += wsel

    @pl.when((e == E - 1) & (hb == HB - 1))
    def _norm():
        out_ref[...] = out_ref[...] / jnp.maximum(tp_ref[...], 1e-8)


def _forward(x, Wg, W1, W2, interpret=False):
    B, T, D = x.shape
    E = Wg.shape[1]
    H = W1.shape[2]
    C = min(T, max(1, int(T * _CAP_FACTOR / E)))
    HB = 4
    HBLK = H // HB
    xb = x.astype(jnp.bfloat16)
    wgb = Wg.astype(jnp.bfloat16)
    w1b = W1.astype(jnp.bfloat16)
    w2b = W2.astype(jnp.bfloat16)
    return pl.pallas_call(
        functools.partial(_router_kernel, E=E, C=C, HB=HB),
        grid=(B, E, HB),
        in_specs=[
            pl.BlockSpec((None, T, D), lambda b, e, hb: (b, 0, 0)),
            pl.BlockSpec((D, E), lambda b, e, hb: (0, 0)),
            pl.BlockSpec((None, D, HBLK), lambda b, e, hb: (e, 0, hb)),
            pl.BlockSpec((None, HBLK, D), lambda b, e, hb: (e, hb, 0)),
        ],
        out_specs=pl.BlockSpec((None, T, D), lambda b, e, hb: (b, 0, 0)),
        out_shape=jax.ShapeDtypeStruct((B, T, D), jnp.float32),
        scratch_shapes=[
            pltpu.VMEM((T, E), jnp.float32),
            pltpu.VMEM((E, T), jnp.float32),
            pltpu.VMEM((T, 1), jnp.float32),
            pltpu.VMEM((T, C), jnp.bfloat16),
            pltpu.VMEM((T, 1), jnp.float32),
            pltpu.VMEM((C, D), jnp.bfloat16),
            pltpu.VMEM((C, D), jnp.float32),
        ],
        compiler_params=pltpu.CompilerParams(
            vmem_limit_bytes=64 * 1024 * 1024),
        interpret=interpret,
    )(xb, wgb, w1b, w2b)


def kernel(x, Wg, W1, W2):
    return _forward(x, Wg, W1, W2)


# speedup vs baseline: 1.2888x; 1.0705x over previous
import functools

import jax
import jax.numpy as jnp
import numpy as np
from jax.experimental import pallas as pl
from jax.experimental.pallas import tpu as pltpu

_CAP_FACTOR = 1.0
_RB = 512


def _gelu_exact(z):
    return 0.5 * z * (1.0 + jax.lax.erf(z * np.float32(1.0 / np.sqrt(2.0))))


def _router_kernel(x_ref, wg_ref, w1_ref, w2_ref, out_ref,
                   wall_ref, wallt_ref, xb_ref, tp_ref, pt_ref, wsel_ref,
                   sel_ref, acc_ref, *, E, C, HB):
    e = pl.program_id(1)
    hb = pl.program_id(2)
    T, D = x_ref.shape

    @pl.when((e == 0) & (hb == 0))
    def _init():
        xb = x_ref[...].astype(jnp.bfloat16)
        xb_ref[...] = xb
        s = jnp.dot(xb, wg_ref[...].astype(jnp.bfloat16),
                    preferred_element_type=jnp.float32)
        m = jnp.max(s, axis=0, keepdims=True)
        ex = jnp.exp(s - m)
        wall = ex / jnp.sum(ex, axis=0, keepdims=True)
        wall_ref[...] = wall
        wallt_ref[...] = wall.T
        tp_ref[...] = jnp.zeros_like(tp_ref)
        out_ref[...] = jnp.zeros_like(out_ref)

    @pl.when(hb == 0)
    def _route():
        mrow = jax.lax.broadcasted_iota(jnp.int32, (1, E), 1) == e
        w_col = jnp.sum(jnp.where(mrow, wall_ref[...], 0.0),
                        axis=1, keepdims=True)
        mcol = jax.lax.broadcasted_iota(jnp.int32, (E, 1), 0) == e
        w_row = jnp.sum(jnp.where(mcol, wallt_ref[...], 0.0),
                        axis=0, keepdims=True)

        jrow = jax.lax.broadcasted_iota(jnp.int32, (1, T), 1)
        cnts = []
        for k in range(T // _RB):
            wi = w_col[k * _RB:(k + 1) * _RB]
            ii = k * _RB + jax.lax.broadcasted_iota(jnp.int32, (_RB, 1), 0)
            ind = ((w_row > wi) | ((w_row == wi) & (jrow < ii))
                   ).astype(jnp.float32)
            cnts.append(jnp.sum(ind, axis=1, keepdims=True))
        rank = jnp.concatenate(cnts, axis=0).astype(jnp.int32)
        wsel_ref[...] = jnp.where(rank < C, w_col, 0.0)

        iota_c = jax.lax.broadcasted_iota(jnp.int32, (T, C), 1)
        pt = (rank == iota_c).astype(jnp.bfloat16)
        pt_ref[...] = pt

        sel_ref[...] = jax.lax.dot_general(
            pt, xb_ref[...], (((0,), (0,)), ((), ())),
            preferred_element_type=jnp.float32).astype(jnp.bfloat16)
        acc_ref[...] = jnp.zeros_like(acc_ref)

    z = jnp.dot(sel_ref[...], w1_ref[...],
                preferred_element_type=jnp.float32)
    h = _gelu_exact(z).astype(jnp.bfloat16)
    acc_ref[...] += jnp.dot(h, w2_ref[...],
                            preferred_element_type=jnp.float32)

    @pl.when(hb == HB - 1)
    def _combine():
        wsel = wsel_ref[...]
        contrib = jnp.dot(pt_ref[...], acc_ref[...].astype(jnp.bfloat16),
                          preferred_element_type=jnp.float32)
        out_ref[...] += contrib * wsel
        tp_ref[...] += wsel

    @pl.when((e == E - 1) & (hb == HB - 1))
    def _norm():
        out_ref[...] = out_ref[...] / jnp.maximum(tp_ref[...], 1e-8)


def _forward(x, Wg, W1, W2, interpret=False):
    B, T, D = x.shape
    E = Wg.shape[1]
    H = W1.shape[2]
    C = min(T, max(1, int(T * _CAP_FACTOR / E)))
    HB = 4
    HBLK = H // HB
    w1b = W1.astype(jnp.bfloat16)
    w2b = W2.astype(jnp.bfloat16)
    return pl.pallas_call(
        functools.partial(_router_kernel, E=E, C=C, HB=HB),
        grid=(B, E, HB),
        in_specs=[
            pl.BlockSpec((None, T, D), lambda b, e, hb: (b, 0, 0)),
            pl.BlockSpec((D, E), lambda b, e, hb: (0, 0)),
            pl.BlockSpec((None, D, HBLK), lambda b, e, hb: (e, 0, hb)),
            pl.BlockSpec((None, HBLK, D), lambda b, e, hb: (e, hb, 0)),
        ],
        out_specs=pl.BlockSpec((None, T, D), lambda b, e, hb: (b, 0, 0)),
        out_shape=jax.ShapeDtypeStruct((B, T, D), jnp.float32),
        scratch_shapes=[
            pltpu.VMEM((T, E), jnp.float32),
            pltpu.VMEM((E, T), jnp.float32),
            pltpu.VMEM((T, D), jnp.bfloat16),
            pltpu.VMEM((T, 1), jnp.float32),
            pltpu.VMEM((T, C), jnp.bfloat16),
            pltpu.VMEM((T, 1), jnp.float32),
            pltpu.VMEM((C, D), jnp.bfloat16),
            pltpu.VMEM((C, D), jnp.float32),
        ],
        interpret=interpret,
    )(x, Wg, w1b, w2b)


def kernel(x, Wg, W1, W2):
    return _forward(x, Wg, W1, W2)
